# baseline (speedup 1.0000x reference)
import functools

import jax
import jax.numpy as jnp
from jax import lax
from jax.experimental import pallas as pl
from jax.experimental.pallas import tpu as pltpu
from jax.experimental.pallas import tpu_sc as plsc

MAX_SEQ_LEN = 200
D_MODEL = 64

NUM_CORES = 2
NUM_SUBCORES = 16
NUM_WORKERS = NUM_CORES * NUM_SUBCORES

LANES = 128
NBUF = 4


def _pos_encoding(seq_len, d_model):
    pos = jnp.arange(seq_len, dtype=jnp.float32)[:, None]
    exp = jnp.arange(0, d_model, 2, dtype=jnp.float32)
    stop = d_model // 2
    pe = jnp.zeros((seq_len, d_model), jnp.float32)
    pe = pe.at[:, 0::2].set(jnp.sin(pos / 10000 ** (exp / d_model)))
    pe = pe.at[:, 1::2].set(jnp.cos(pos / 10000 ** (exp[:stop] / d_model)))
    return pe


@jax.jit
def _embed(xv, table, pe):
    st_n, bt_n, ss_n, _ = xv.shape
    seq_len = st_n * ss_n
    n_tiles = seq_len * bt_n
    per_w = n_tiles // NUM_WORKERS
    n_outer = per_w // NBUF
    dt_n = D_MODEL // 8
    mesh = plsc.VectorSubcoreMesh(core_axis_name="c", subcore_axis_name="s")

    @functools.partial(
        pl.kernel,
        mesh=mesh,
        out_type=jax.ShapeDtypeStruct(
            (seq_len, dt_n, bt_n, 8, LANES), jnp.float32
        ),
        scratch_types=(
            [pltpu.VMEM((LANES,), jnp.int32)] * NBUF
            + [pltpu.VMEM((LANES, D_MODEL), jnp.float32)] * NBUF
            + [pltpu.VMEM((dt_n, 8, LANES + 1), jnp.float32)] * NBUF
            + [pltpu.VMEM((MAX_SEQ_LEN, D_MODEL), jnp.float32)]
            + [pltpu.SemaphoreType.DMA] * (3 * NBUF)
        ),
        compiler_params=pltpu.CompilerParams(
            use_tc_tiling_on_sc=False, needs_layout_passes=False
        ),
    )
    def k(xv_hbm, tab_hbm, pe_hbm, out_hbm, *scratch):
        idx = list(scratch[0:NBUF])
        rows = list(scratch[NBUF:2 * NBUF])
        tr = list(scratch[2 * NBUF:3 * NBUF])
        pe_v = scratch[3 * NBUF]
        sg = list(scratch[3 * NBUF + 1:3 * NBUF + 1 + NBUF])
        so = list(scratch[3 * NBUF + 1 + NBUF:3 * NBUF + 1 + 2 * NBUF])
        si = list(scratch[3 * NBUF + 1 + 2 * NBUF:3 * NBUF + 1 + 3 * NBUF])

        wid = lax.axis_index("s") * NUM_CORES + lax.axis_index("c")
        base = wid * per_w

        pltpu.sync_copy(pe_hbm, pe_v)

        lane_iota = lax.iota(jnp.int32, 16)
        dt_idx = [
            lax.shift_right_logical(lane_iota + (g * 16), 3)
            for g in range(D_MODEL // 16)
        ]
        ds_idx = [
            lax.bitwise_and(lane_iota + (g * 16), 7)
            for g in range(D_MODEL // 16)
        ]

        def coords(t):
            g = base + t
            s = lax.shift_right_logical(g, 5)
            bt = lax.bitwise_and(g, bt_n - 1)
            return s, bt

        def fire_idx(t, b):
            s, bt = coords(t)
            pltpu.async_copy(
                xv_hbm.at[lax.shift_right_logical(s, 3), bt,
                          lax.bitwise_and(s, ss_n - 1)],
                idx[b], si[b],
            )

        def wait_idx(b):
            pltpu.make_async_copy(xv_hbm.at[0, 0, 0], idx[b], si[b]).wait()

        def fire_gather(b):
            pltpu.async_copy(tab_hbm.at[idx[b]], rows[b], sg[b])

        def wait_gather(b):
            pltpu.make_async_copy(tab_hbm.at[idx[b]], rows[b], sg[b]).wait()

        def fire_out(s, bt, b):
            pltpu.async_copy(
                tr[b].at[:, :, pl.ds(0, LANES)], out_hbm.at[s, :, bt], so[b]
            )

        def wait_out(b):
            pltpu.make_async_copy(
                tr[b].at[:, :, pl.ds(0, LANES)], out_hbm.at[0, :, 0], so[b]
            ).wait()

        def transpose_add(s, b):
            pe_g = [pe_v[s, pl.ds(g * 16, 16)] for g in range(D_MODEL // 16)]

            @plsc.parallel_loop(0, LANES, step=1, unroll=4)
            def _(tok):
                col = jnp.full((16,), 0, jnp.int32) + tok
                for g in range(D_MODEL // 16):
                    val = rows[b][tok, pl.ds(g * 16, 16)] + pe_g[g]
                    plsc.store_scatter(
                        tr[b], [dt_idx[g], ds_idx[g], col], val
                    )

        for b in range(NBUF - 1):
            fire_idx(b, b)
        for b in range(NBUF - 1):
            wait_idx(b)
            fire_gather(b)
        fire_idx(NBUF - 1, NBUF - 1)

        def outer(go, carry):
            for b in range(NBUF):
                t = go * NBUF + b
                b3 = (b + NBUF - 1) % NBUF
                s, bt = coords(t)
                wait_gather(b)

                def prefetch():
                    wait_idx(b3)
                    fire_gather(b3)

                def prefetch_idx():
                    fire_idx(t + NBUF, b)

                if b == 0:
                    prefetch()

                    @pl.when(go < n_outer - 1)
                    def _():
                        prefetch_idx()
                else:
                    @pl.when(go < n_outer - 1)
                    def _():
                        prefetch()
                        prefetch_idx()

                @pl.when(go > 0)
                def _():
                    wait_out(b)

                transpose_add(s, b)
                fire_out(s, bt, b)
            return carry

        lax.fori_loop(0, n_outer, outer, 0)
        for b in range(NBUF):
            wait_out(b)

    return k(xv, table, pe)


def kernel(x, table):
    batch, seq_len = x.shape
    pe = _pos_encoding(seq_len, D_MODEL)
    xv = (
        x.T.reshape(seq_len // 8, 8, batch // LANES, LANES)
        .transpose(0, 2, 1, 3)
    )
    out5 = _embed(xv, table, pe)
    out = (
        out5.transpose(2, 4, 0, 1, 3)
        .reshape(batch, seq_len, D_MODEL)
    )
    return out


# speedup vs baseline: 1.0039x; 1.0039x over previous
import functools

import jax
import jax.numpy as jnp
from jax import lax
from jax.experimental import pallas as pl
from jax.experimental.pallas import tpu as pltpu
from jax.experimental.pallas import tpu_sc as plsc

MAX_SEQ_LEN = 200
D_MODEL = 64

NUM_CORES = 2
NUM_SUBCORES = 16
NUM_WORKERS = NUM_CORES * NUM_SUBCORES

LANES = 128
NBUF = 4


def _pos_encoding(seq_len, d_model):
    pos = jnp.arange(seq_len, dtype=jnp.float32)[:, None]
    exp = jnp.arange(0, d_model, 2, dtype=jnp.float32)
    stop = d_model // 2
    pe = jnp.zeros((seq_len, d_model), jnp.float32)
    pe = pe.at[:, 0::2].set(jnp.sin(pos / 10000 ** (exp / d_model)))
    pe = pe.at[:, 1::2].set(jnp.cos(pos / 10000 ** (exp[:stop] / d_model)))
    return pe


@jax.jit
def _embed(xv, table, pe):
    st_n, bt_n, ss_n, _ = xv.shape
    seq_len = st_n * ss_n
    n_tiles = seq_len * bt_n
    per_w = n_tiles // NUM_WORKERS
    n_outer = per_w // NBUF
    dt_n = D_MODEL // 8
    mesh = plsc.VectorSubcoreMesh(core_axis_name="c", subcore_axis_name="s")

    @functools.partial(
        pl.kernel,
        mesh=mesh,
        out_type=jax.ShapeDtypeStruct(
            (seq_len, dt_n, bt_n, 8, LANES), jnp.float32
        ),
        scratch_types=(
            [pltpu.VMEM((LANES,), jnp.int32)] * NBUF
            + [pltpu.VMEM((LANES, D_MODEL), jnp.float32)] * NBUF
            + [pltpu.VMEM((dt_n, 8, LANES + 1), jnp.float32)] * NBUF
            + [pltpu.VMEM((MAX_SEQ_LEN, D_MODEL), jnp.float32)]
            + [pltpu.SemaphoreType.DMA] * (3 * NBUF)
        ),
        compiler_params=pltpu.CompilerParams(
            use_tc_tiling_on_sc=False, needs_layout_passes=False
        ),
    )
    def k(xv_hbm, tab_hbm, pe_hbm, out_hbm, *scratch):
        idx = list(scratch[0:NBUF])
        rows = list(scratch[NBUF:2 * NBUF])
        tr = list(scratch[2 * NBUF:3 * NBUF])
        pe_v = scratch[3 * NBUF]
        sg = list(scratch[3 * NBUF + 1:3 * NBUF + 1 + NBUF])
        so = list(scratch[3 * NBUF + 1 + NBUF:3 * NBUF + 1 + 2 * NBUF])
        si = list(scratch[3 * NBUF + 1 + 2 * NBUF:3 * NBUF + 1 + 3 * NBUF])

        wid = lax.axis_index("s") * NUM_CORES + lax.axis_index("c")
        base = wid * per_w

        pltpu.sync_copy(pe_hbm, pe_v)

        lane_iota = lax.iota(jnp.int32, 16)
        dt_idx = [
            lax.shift_right_logical(lane_iota + (g * 16), 3)
            for g in range(D_MODEL // 16)
        ]
        ds_idx = [
            lax.bitwise_and(lane_iota + (g * 16), 7)
            for g in range(D_MODEL // 16)
        ]

        def coords(t):
            g = base + t
            s = lax.shift_right_logical(g, 5)
            bt = lax.bitwise_and(g, bt_n - 1)
            return s, bt

        def fire_idx(t, b):
            s, bt = coords(t)
            pltpu.async_copy(
                xv_hbm.at[lax.shift_right_logical(s, 3), bt,
                          lax.bitwise_and(s, ss_n - 1)],
                idx[b], si[b],
            )

        def wait_idx(b):
            pltpu.make_async_copy(xv_hbm.at[0, 0, 0], idx[b], si[b]).wait()

        def fire_gather(b):
            pltpu.async_copy(tab_hbm.at[idx[b]], rows[b], sg[b])

        def wait_gather(b):
            pltpu.make_async_copy(tab_hbm.at[idx[b]], rows[b], sg[b]).wait()

        def fire_out(s, bt, b):
            pltpu.async_copy(
                tr[b].at[:, :, pl.ds(0, LANES)], out_hbm.at[s, :, bt], so[b]
            )

        def wait_out(b):
            pltpu.make_async_copy(
                tr[b].at[:, :, pl.ds(0, LANES)], out_hbm.at[0, :, 0], so[b]
            ).wait()

        def transpose_add(s, b):
            pe_g = [pe_v[s, pl.ds(g * 16, 16)] for g in range(D_MODEL // 16)]

            @plsc.parallel_loop(0, LANES, step=1, unroll=4)
            def _(tok):
                col = jnp.full((16,), 0, jnp.int32) + tok
                for g in range(D_MODEL // 16):
                    val = rows[b][tok, pl.ds(g * 16, 16)] + pe_g[g]
                    plsc.store_scatter(
                        tr[b], [dt_idx[g], ds_idx[g], col], val
                    )

        for b in range(NBUF - 1):
            fire_idx(b, b)
        for b in range(NBUF - 1):
            wait_idx(b)
            fire_gather(b)
        fire_idx(NBUF - 1, NBUF - 1)

        def outer(go, carry):
            for b in range(NBUF):
                t = go * NBUF + b
                b3 = (b + NBUF - 1) % NBUF
                s, bt = coords(t)
                wait_gather(b)

                def prefetch():
                    wait_idx(b3)
                    fire_gather(b3)

                def prefetch_idx():
                    fire_idx(t + NBUF, b)

                if b == 0:
                    prefetch()

                    @pl.when(go < n_outer - 1)
                    def _():
                        prefetch_idx()
                else:
                    @pl.when(go < n_outer - 1)
                    def _():
                        prefetch()
                        prefetch_idx()

                @pl.when(go > 0)
                def _():
                    wait_out(b)

                transpose_add(s, b)
                fire_out(s, bt, b)
            return carry

        lax.fori_loop(0, n_outer, outer, 0)
        for b in range(NBUF):
            wait_out(b)

    return k(xv, table, pe)


def kernel(x, table):
    batch, seq_len = x.shape
    pe = _pos_encoding(seq_len, D_MODEL)
    xv = (
        x.astype(jnp.int32).T
        .reshape(seq_len // 8, 8, batch // LANES, LANES)
        .transpose(0, 2, 1, 3)
    )
    out5 = _embed(xv, table, pe)
    out = (
        out5.transpose(2, 4, 0, 1, 3)
        .reshape(batch, seq_len, D_MODEL)
    )
    return out
